# baseline (speedup 1.0000x reference)
import functools

import jax
import jax.numpy as jnp
from jax import lax
from jax.experimental import pallas as pl
from jax.experimental.pallas import tpu as pltpu
from jax.experimental.pallas import tpu_sc as plsc

N_ROWS = 320000
D_FEAT = 128
N_SEG = 10000

NC = 2
NS = 16
NW = NC * NS
ROWS_PER_TILE = N_ROWS // NW
CHUNK = 96
NSLOT = 4
NMAIN = 104
NTRI = NMAIN // NSLOT
TAIL = ROWS_PER_TILE - NMAIN * CHUNK
ZROWS = 624
ZROWS_LAST = N_SEG - (NS - 1) * ZROWS
ZBLK = 48
LANES = 16

_mesh = plsc.VectorSubcoreMesh(core_axis_name="c", subcore_axis_name="s")


@functools.partial(
    pl.kernel,
    mesh=_mesh,
    out_type=jax.ShapeDtypeStruct((NC, N_SEG, D_FEAT), jnp.float32),
    scratch_types=[
        pltpu.VMEM((CHUNK,), jnp.int32),
        pltpu.VMEM((CHUNK,), jnp.int32),
        pltpu.VMEM((CHUNK,), jnp.int32),
        pltpu.VMEM((CHUNK,), jnp.int32),
        pltpu.VMEM((TAIL,), jnp.int32),
        pltpu.VMEM((NSLOT * CHUNK, D_FEAT), jnp.float32),
        pltpu.VMEM_SHARED((N_SEG, D_FEAT), jnp.float32),
        pltpu.SemaphoreType.DMA,
        pltpu.SemaphoreType.DMA,
        pltpu.SemaphoreType.DMA,
        pltpu.SemaphoreType.DMA,
    ],
)
def _segment_sum_sc(data_hbm, seg_hbm, out_hbm,
                    i0, i1, i2, i3, ids_t, ring, acc, l0, l1, l2, l3):
    c = lax.axis_index("c")
    s = lax.axis_index("s")
    wid = c * NS + s
    base0 = wid * ROWS_PER_TILE
    idbufs = (i0, i1, i2, i3)
    lsems = (l0, l1, l2, l3)

    def slot(k):
        return ring.at[pl.ds(k * CHUNK, CHUNK)]

    zv = jnp.zeros((LANES,), jnp.float32)

    def zrow(r, carry):
        for g in range(D_FEAT // LANES):
            ring[r, pl.ds(g * LANES, LANES)] = zv
        return carry

    lax.fori_loop(0, ZBLK, zrow, 0)
    for z in range(ZROWS // ZBLK):
        pltpu.sync_copy(ring.at[pl.ds(0, ZBLK)],
                        acc.at[pl.ds(s * ZROWS + z * ZBLK, ZBLK)])

    @pl.when(s == NS - 1)
    def _():
        pltpu.sync_copy(ring.at[pl.ds(0, ZROWS_LAST - ZROWS)],
                        acc.at[pl.ds(N_SEG - (ZROWS_LAST - ZROWS),
                                     ZROWS_LAST - ZROWS)])

    def load(j, k):
        pltpu.async_copy(data_hbm.at[pl.ds(base0 + j * CHUNK, CHUNK)],
                         slot(k), lsems[k])
        pltpu.async_copy(seg_hbm.at[pl.ds(base0 + j * CHUNK, CHUNK)],
                         idbufs[k], lsems[k])

    def wait_load(j, k):
        pltpu.make_async_copy(data_hbm.at[pl.ds(base0 + j * CHUNK, CHUNK)],
                              slot(k), lsems[k]).wait()
        pltpu.make_async_copy(seg_hbm.at[pl.ds(base0 + j * CHUNK, CHUNK)],
                              idbufs[k], lsems[k]).wait()

    pltpu.sync_copy(seg_hbm.at[pl.ds(base0 + NMAIN * CHUNK, TAIL)], ids_t)
    load(0, 0)
    load(1, 1)
    load(2, 2)
    plsc.subcore_barrier()

    def body(t, carry):
        c0 = NSLOT * t
        for k in range(NSLOT):
            ch = c0 + k
            k2 = (k + 3) % NSLOT
            wait_load(ch, k)
            if k == 0:
                load(ch + 3, k2)
            else:
                @pl.when(t < NTRI - 1)
                def _():
                    load(ch + 3, k2)
            pltpu.sync_copy(slot(k), acc.at[idbufs[k]], add=True)
        return carry

    lax.fori_loop(0, NTRI, body, 0)

    pltpu.sync_copy(data_hbm.at[pl.ds(base0 + NMAIN * CHUNK, TAIL)],
                    ring.at[pl.ds(0, TAIL)])
    pltpu.sync_copy(ring.at[pl.ds(0, TAIL)], acc.at[ids_t], add=True)

    plsc.subcore_barrier()

    @pl.when(s < NS - 1)
    def _():
        pltpu.sync_copy(acc.at[pl.ds(s * ZROWS, ZROWS)],
                        out_hbm.at[c, pl.ds(s * ZROWS, ZROWS)])

    @pl.when(s == NS - 1)
    def _():
        pltpu.sync_copy(acc.at[pl.ds((NS - 1) * ZROWS, ZROWS_LAST)],
                        out_hbm.at[c, pl.ds((NS - 1) * ZROWS, ZROWS_LAST)])


def _combine_body(p_ref, o_ref):
    o_ref[...] = p_ref[0] + p_ref[1]


def _combine(partials):
    nblk = 1
    rows = N_SEG // nblk
    return pl.pallas_call(
        _combine_body,
        out_shape=jax.ShapeDtypeStruct((N_SEG, D_FEAT), jnp.float32),
        grid=(nblk,),
        in_specs=[pl.BlockSpec((NC, rows, D_FEAT), lambda i: (0, i, 0))],
        out_specs=pl.BlockSpec((rows, D_FEAT), lambda i: (i, 0)),
    )(partials)


def kernel(data, segment_ids):
    seg = segment_ids.astype(jnp.int32)
    partials = _segment_sum_sc(data, seg)
    return _combine(partials)


# speedup vs baseline: 1.0133x; 1.0133x over previous
import functools

import jax
import jax.numpy as jnp
from jax import lax
from jax.experimental import pallas as pl
from jax.experimental.pallas import tpu as pltpu
from jax.experimental.pallas import tpu_sc as plsc

N_ROWS = 320000
D_FEAT = 128
N_SEG = 10000

NC = 2
NS = 16
NW = NC * NS
ROWS_PER_TILE = N_ROWS // NW
CHUNK = 96
NSLOT = 4
NMAIN = 104
NTRI = NMAIN // NSLOT
TAIL = ROWS_PER_TILE - NMAIN * CHUNK
ZROWS = 624
ZROWS_LAST = N_SEG - (NS - 1) * ZROWS
ZBLK = 48
LANES = 16

_mesh = plsc.VectorSubcoreMesh(core_axis_name="c", subcore_axis_name="s")


@functools.partial(
    pl.kernel,
    mesh=_mesh,
    out_type=jax.ShapeDtypeStruct((NC, N_SEG, D_FEAT), jnp.float32),
    scratch_types=[
        pltpu.VMEM((CHUNK,), jnp.int32),
        pltpu.VMEM((CHUNK,), jnp.int32),
        pltpu.VMEM((CHUNK,), jnp.int32),
        pltpu.VMEM((CHUNK,), jnp.int32),
        pltpu.VMEM((TAIL,), jnp.int32),
        pltpu.VMEM((NSLOT * CHUNK, D_FEAT), jnp.float32),
        pltpu.VMEM_SHARED((N_SEG, D_FEAT), jnp.float32),
        pltpu.SemaphoreType.DMA,
        pltpu.SemaphoreType.DMA,
        pltpu.SemaphoreType.DMA,
        pltpu.SemaphoreType.DMA,
    ],
)
def _segment_sum_sc(data_hbm, seg_hbm, out_hbm,
                    i0, i1, i2, i3, ids_t, ring, acc, l0, l1, l2, l3):
    c = lax.axis_index("c")
    s = lax.axis_index("s")
    wid = c * NS + s
    base0 = wid * ROWS_PER_TILE
    idbufs = (i0, i1, i2, i3)
    lsems = (l0, l1, l2, l3)

    def slot(k):
        return ring.at[pl.ds(k * CHUNK, CHUNK)]

    zv = jnp.zeros((LANES,), jnp.float32)

    def zrow(r, carry):
        for g in range(D_FEAT // LANES):
            ring[r, pl.ds(g * LANES, LANES)] = zv
        return carry

    lax.fori_loop(0, ZBLK, zrow, 0)
    for z in range(ZROWS // ZBLK):
        pltpu.sync_copy(ring.at[pl.ds(0, ZBLK)],
                        acc.at[pl.ds(s * ZROWS + z * ZBLK, ZBLK)])

    @pl.when(s == NS - 1)
    def _():
        pltpu.sync_copy(ring.at[pl.ds(0, ZROWS_LAST - ZROWS)],
                        acc.at[pl.ds(N_SEG - (ZROWS_LAST - ZROWS),
                                     ZROWS_LAST - ZROWS)])

    def load(j, k):
        pltpu.async_copy(data_hbm.at[pl.ds(base0 + j * CHUNK, CHUNK)],
                         slot(k), lsems[k])
        pltpu.async_copy(seg_hbm.at[pl.ds(base0 + j * CHUNK, CHUNK)],
                         idbufs[k], lsems[k])

    def wait_load(j, k):
        pltpu.make_async_copy(data_hbm.at[pl.ds(base0 + j * CHUNK, CHUNK)],
                              slot(k), lsems[k]).wait()
        pltpu.make_async_copy(seg_hbm.at[pl.ds(base0 + j * CHUNK, CHUNK)],
                              idbufs[k], lsems[k]).wait()

    pltpu.sync_copy(seg_hbm.at[pl.ds(base0 + NMAIN * CHUNK, TAIL)], ids_t)
    load(0, 0)
    load(1, 1)
    load(2, 2)
    plsc.subcore_barrier()

    def body(t, carry):
        c0 = NSLOT * t
        for k in range(NSLOT):
            ch = c0 + k
            k2 = (k + 3) % NSLOT
            wait_load(ch, k)
            if k == 0:
                load(ch + 3, k2)
            else:
                @pl.when(t < NTRI - 1)
                def _():
                    load(ch + 3, k2)
            pltpu.sync_copy(slot(k), acc.at[idbufs[k]], add=True)
        return carry

    lax.fori_loop(0, NTRI, body, 0)

    pltpu.sync_copy(data_hbm.at[pl.ds(base0 + NMAIN * CHUNK, TAIL)],
                    ring.at[pl.ds(0, TAIL)])
    pltpu.sync_copy(ring.at[pl.ds(0, TAIL)], acc.at[ids_t], add=True)

    plsc.subcore_barrier()

    @pl.when(s < NS - 1)
    def _():
        pltpu.sync_copy(acc.at[pl.ds(s * ZROWS, ZROWS)],
                        out_hbm.at[c, pl.ds(s * ZROWS, ZROWS)])

    @pl.when(s == NS - 1)
    def _():
        pltpu.sync_copy(acc.at[pl.ds((NS - 1) * ZROWS, ZROWS_LAST)],
                        out_hbm.at[c, pl.ds((NS - 1) * ZROWS, ZROWS_LAST)])


def _combine_body(p_ref, o_ref):
    o_ref[...] = p_ref[0] + p_ref[1]


def _combine(partials):
    nblk = 2
    rows = N_SEG // nblk
    return pl.pallas_call(
        _combine_body,
        out_shape=jax.ShapeDtypeStruct((N_SEG, D_FEAT), jnp.float32),
        grid=(nblk,),
        in_specs=[pl.BlockSpec((NC, rows, D_FEAT), lambda i: (0, i, 0))],
        out_specs=pl.BlockSpec((rows, D_FEAT), lambda i: (i, 0)),
    )(partials)


def kernel(data, segment_ids):
    seg = segment_ids.astype(jnp.int32)
    partials = _segment_sum_sc(data, seg)
    return _combine(partials)


# speedup vs baseline: 1.0213x; 1.0079x over previous
import functools

import jax
import jax.numpy as jnp
from jax import lax
from jax.experimental import pallas as pl
from jax.experimental.pallas import tpu as pltpu
from jax.experimental.pallas import tpu_sc as plsc

N_ROWS = 320000
D_FEAT = 128
N_SEG = 10000

NC = 2
NS = 16
NW = NC * NS
ROWS_PER_TILE = N_ROWS // NW
CHUNK = 96
NSLOT = 4
NMAIN = 104
NTRI = NMAIN // NSLOT
TAIL = ROWS_PER_TILE - NMAIN * CHUNK
ZROWS = 624
ZROWS_LAST = N_SEG - (NS - 1) * ZROWS
ZBLK = 104
LANES = 16

_mesh = plsc.VectorSubcoreMesh(core_axis_name="c", subcore_axis_name="s")


@functools.partial(
    pl.kernel,
    mesh=_mesh,
    out_type=jax.ShapeDtypeStruct((NC, N_SEG, D_FEAT), jnp.float32),
    scratch_types=[
        pltpu.VMEM((CHUNK,), jnp.int32),
        pltpu.VMEM((CHUNK,), jnp.int32),
        pltpu.VMEM((CHUNK,), jnp.int32),
        pltpu.VMEM((CHUNK,), jnp.int32),
        pltpu.VMEM((TAIL,), jnp.int32),
        pltpu.VMEM((NSLOT * CHUNK, D_FEAT), jnp.float32),
        pltpu.VMEM_SHARED((N_SEG, D_FEAT), jnp.float32),
        pltpu.SemaphoreType.DMA,
        pltpu.SemaphoreType.DMA,
        pltpu.SemaphoreType.DMA,
        pltpu.SemaphoreType.DMA,
    ],
)
def _segment_sum_sc(data_hbm, seg_hbm, out_hbm,
                    i0, i1, i2, i3, ids_t, ring, acc, l0, l1, l2, l3):
    c = lax.axis_index("c")
    s = lax.axis_index("s")
    wid = c * NS + s
    base0 = wid * ROWS_PER_TILE
    idbufs = (i0, i1, i2, i3)
    lsems = (l0, l1, l2, l3)

    def slot(k):
        return ring.at[pl.ds(k * CHUNK, CHUNK)]

    def load(j, k):
        pltpu.async_copy(data_hbm.at[pl.ds(base0 + j * CHUNK, CHUNK)],
                         slot(k), lsems[k])
        pltpu.async_copy(seg_hbm.at[pl.ds(base0 + j * CHUNK, CHUNK)],
                         idbufs[k], lsems[k])

    def wait_load(j, k):
        pltpu.make_async_copy(data_hbm.at[pl.ds(base0 + j * CHUNK, CHUNK)],
                              slot(k), lsems[k]).wait()
        pltpu.make_async_copy(seg_hbm.at[pl.ds(base0 + j * CHUNK, CHUNK)],
                              idbufs[k], lsems[k]).wait()

    pltpu.sync_copy(seg_hbm.at[pl.ds(base0 + NMAIN * CHUNK, TAIL)], ids_t)
    load(0, 0)
    load(1, 1)
    load(2, 2)

    zv = jnp.zeros((LANES,), jnp.float32)
    zbase = NSLOT * CHUNK - ZBLK

    def zrow(r, carry):
        for g in range(D_FEAT // LANES):
            ring[zbase + r, pl.ds(g * LANES, LANES)] = zv
        return carry

    lax.fori_loop(0, ZBLK, zrow, 0)
    zsrc = ring.at[pl.ds(zbase, ZBLK)]
    for z in range(ZROWS // ZBLK):
        pltpu.async_copy(zsrc, acc.at[pl.ds(s * ZROWS + z * ZBLK, ZBLK)], l3)

    @pl.when(s == NS - 1)
    def _():
        pltpu.sync_copy(ring.at[pl.ds(zbase, ZROWS_LAST - ZROWS)],
                        acc.at[pl.ds(N_SEG - (ZROWS_LAST - ZROWS),
                                     ZROWS_LAST - ZROWS)])

    for z in range(ZROWS // ZBLK):
        pltpu.make_async_copy(
            zsrc, acc.at[pl.ds(s * ZROWS + z * ZBLK, ZBLK)], l3).wait()
    plsc.subcore_barrier()

    def body(t, carry):
        c0 = NSLOT * t
        for k in range(NSLOT):
            ch = c0 + k
            k2 = (k + 3) % NSLOT
            wait_load(ch, k)
            if k == 0:
                load(ch + 3, k2)
            else:
                @pl.when(t < NTRI - 1)
                def _():
                    load(ch + 3, k2)
            pltpu.sync_copy(slot(k), acc.at[idbufs[k]], add=True)
        return carry

    lax.fori_loop(0, NTRI, body, 0)

    pltpu.sync_copy(data_hbm.at[pl.ds(base0 + NMAIN * CHUNK, TAIL)],
                    ring.at[pl.ds(0, TAIL)])
    pltpu.sync_copy(ring.at[pl.ds(0, TAIL)], acc.at[ids_t], add=True)

    plsc.subcore_barrier()

    @pl.when(s < NS - 1)
    def _():
        pltpu.sync_copy(acc.at[pl.ds(s * ZROWS, ZROWS)],
                        out_hbm.at[c, pl.ds(s * ZROWS, ZROWS)])

    @pl.when(s == NS - 1)
    def _():
        pltpu.sync_copy(acc.at[pl.ds((NS - 1) * ZROWS, ZROWS_LAST)],
                        out_hbm.at[c, pl.ds((NS - 1) * ZROWS, ZROWS_LAST)])


def _combine_body(p_ref, o_ref):
    o_ref[...] = p_ref[0] + p_ref[1]


def _combine(partials):
    nblk = 2
    rows = N_SEG // nblk
    return pl.pallas_call(
        _combine_body,
        out_shape=jax.ShapeDtypeStruct((N_SEG, D_FEAT), jnp.float32),
        grid=(nblk,),
        in_specs=[pl.BlockSpec((NC, rows, D_FEAT), lambda i: (0, i, 0))],
        out_specs=pl.BlockSpec((rows, D_FEAT), lambda i: (i, 0)),
    )(partials)


def kernel(data, segment_ids):
    seg = segment_ids.astype(jnp.int32)
    partials = _segment_sum_sc(data, seg)
    return _combine(partials)


# speedup vs baseline: 1.0241x; 1.0028x over previous
import functools

import jax
import jax.numpy as jnp
from jax import lax
from jax.experimental import pallas as pl
from jax.experimental.pallas import tpu as pltpu
from jax.experimental.pallas import tpu_sc as plsc

N_ROWS = 320000
D_FEAT = 128
N_SEG = 10000

NC = 2
NS = 16
NW = NC * NS
ROWS_PER_TILE = N_ROWS // NW
CHUNK = 96
NSLOT = 4
NMAIN = 104
NTRI = NMAIN // NSLOT
TAIL = ROWS_PER_TILE - NMAIN * CHUNK
ZROWS = 624
ZROWS_LAST = N_SEG - (NS - 1) * ZROWS
ZBLK = 96
LANES = 16

_mesh = plsc.VectorSubcoreMesh(core_axis_name="c", subcore_axis_name="s")


@functools.partial(
    pl.kernel,
    mesh=_mesh,
    out_type=jax.ShapeDtypeStruct((NC, N_SEG, D_FEAT), jnp.float32),
    scratch_types=[
        pltpu.VMEM((CHUNK,), jnp.int32),
        pltpu.VMEM((CHUNK,), jnp.int32),
        pltpu.VMEM((CHUNK,), jnp.int32),
        pltpu.VMEM((CHUNK,), jnp.int32),
        pltpu.VMEM((TAIL,), jnp.int32),
        pltpu.VMEM((NSLOT * CHUNK, D_FEAT), jnp.float32),
        pltpu.VMEM_SHARED((N_SEG, D_FEAT), jnp.float32),
        pltpu.SemaphoreType.DMA,
        pltpu.SemaphoreType.DMA,
        pltpu.SemaphoreType.DMA,
        pltpu.SemaphoreType.DMA,
    ],
)
def _segment_sum_sc(data_hbm, seg_hbm, out_hbm,
                    i0, i1, i2, i3, ids_t, ring, acc, l0, l1, l2, l3):
    c = lax.axis_index("c")
    s = lax.axis_index("s")
    wid = c * NS + s
    base0 = wid * ROWS_PER_TILE
    idbufs = (i0, i1, i2, i3)
    lsems = (l0, l1, l2, l3)

    def slot(k):
        return ring.at[pl.ds(k * CHUNK, CHUNK)]

    def load(j, k):
        pltpu.async_copy(data_hbm.at[pl.ds(base0 + j * CHUNK, CHUNK)],
                         slot(k), lsems[k])
        pltpu.async_copy(seg_hbm.at[pl.ds(base0 + j * CHUNK, CHUNK)],
                         idbufs[k], lsems[k])

    def wait_load(j, k):
        pltpu.make_async_copy(data_hbm.at[pl.ds(base0 + j * CHUNK, CHUNK)],
                              slot(k), lsems[k]).wait()
        pltpu.make_async_copy(seg_hbm.at[pl.ds(base0 + j * CHUNK, CHUNK)],
                              idbufs[k], lsems[k]).wait()

    pltpu.sync_copy(seg_hbm.at[pl.ds(base0 + NMAIN * CHUNK, TAIL)], ids_t)
    load(0, 0)
    load(1, 1)
    load(2, 2)

    zv = jnp.zeros((LANES,), jnp.float32)
    zbase = NSLOT * CHUNK - ZBLK

    def zrow(r, carry):
        for g in range(D_FEAT // LANES):
            ring[zbase + r, pl.ds(g * LANES, LANES)] = zv
        return carry

    lax.fori_loop(0, ZBLK, zrow, 0)
    zsrc = ring.at[pl.ds(zbase, ZBLK)]
    zrem = ZROWS - (ZROWS // ZBLK) * ZBLK
    zsrc_r = ring.at[pl.ds(zbase, zrem)]
    rbase = (ZROWS // ZBLK) * ZBLK
    for z in range(ZROWS // ZBLK):
        pltpu.async_copy(zsrc, acc.at[pl.ds(s * ZROWS + z * ZBLK, ZBLK)], l3)
    pltpu.async_copy(zsrc_r, acc.at[pl.ds(s * ZROWS + rbase, zrem)], l3)

    @pl.when(s == NS - 1)
    def _():
        pltpu.sync_copy(ring.at[pl.ds(zbase, ZROWS_LAST - ZROWS)],
                        acc.at[pl.ds(N_SEG - (ZROWS_LAST - ZROWS),
                                     ZROWS_LAST - ZROWS)])

    for z in range(ZROWS // ZBLK):
        pltpu.make_async_copy(
            zsrc, acc.at[pl.ds(s * ZROWS + z * ZBLK, ZBLK)], l3).wait()
    pltpu.make_async_copy(
        zsrc_r, acc.at[pl.ds(s * ZROWS + rbase, zrem)], l3).wait()
    plsc.subcore_barrier()

    def body(t, carry):
        c0 = NSLOT * t
        for k in range(NSLOT):
            ch = c0 + k
            k2 = (k + 3) % NSLOT
            wait_load(ch, k)
            if k == 0:
                load(ch + 3, k2)
            else:
                @pl.when(t < NTRI - 1)
                def _():
                    load(ch + 3, k2)
            pltpu.sync_copy(slot(k), acc.at[idbufs[k]], add=True)
        return carry

    lax.fori_loop(0, NTRI, body, 0)

    pltpu.sync_copy(data_hbm.at[pl.ds(base0 + NMAIN * CHUNK, TAIL)],
                    ring.at[pl.ds(0, TAIL)])
    pltpu.sync_copy(ring.at[pl.ds(0, TAIL)], acc.at[ids_t], add=True)

    plsc.subcore_barrier()

    @pl.when(s < NS - 1)
    def _():
        pltpu.sync_copy(acc.at[pl.ds(s * ZROWS, ZROWS)],
                        out_hbm.at[c, pl.ds(s * ZROWS, ZROWS)])

    @pl.when(s == NS - 1)
    def _():
        pltpu.sync_copy(acc.at[pl.ds((NS - 1) * ZROWS, ZROWS_LAST)],
                        out_hbm.at[c, pl.ds((NS - 1) * ZROWS, ZROWS_LAST)])


def _combine_body(p_ref, o_ref):
    o_ref[...] = p_ref[0] + p_ref[1]


def _combine(partials):
    nblk = 2
    rows = N_SEG // nblk
    return pl.pallas_call(
        _combine_body,
        out_shape=jax.ShapeDtypeStruct((N_SEG, D_FEAT), jnp.float32),
        grid=(nblk,),
        in_specs=[pl.BlockSpec((NC, rows, D_FEAT), lambda i: (0, i, 0))],
        out_specs=pl.BlockSpec((rows, D_FEAT), lambda i: (i, 0)),
    )(partials)


def kernel(data, segment_ids):
    seg = segment_ids.astype(jnp.int32)
    partials = _segment_sum_sc(data, seg)
    return _combine(partials)
